# baseline (speedup 1.0000x reference)
import functools

import jax
import jax.numpy as jnp
from jax import lax
from jax.experimental import pallas as pl
from jax.experimental.pallas import tpu as pltpu
from jax.experimental.pallas import tpu_sc as plsc

B = 16384
D = 64
MARGIN = 4.0
NC = 2
NS = 16
NW = NC * NS
CB = B // NW
KB = 64
LN = 16
NV = 4
NG = CB // LN

_mesh = plsc.VectorSubcoreMesh(
    core_axis_name="c", subcore_axis_name="s", num_cores=NC, num_subcores=NS
)
_params = pltpu.CompilerParams(needs_layout_passes=False)


def _splat(x, dtype=None):
    v = jnp.broadcast_to(x, (LN,))
    return v if dtype is None else v.astype(dtype)


def _rsqrt(x):
    i = plsc.bitcast(x, jnp.int32)
    y = plsc.bitcast(jnp.int32(0x5F3759DF) - (i >> 1), jnp.float32)
    for _ in range(3):
        y = y * (1.5 - 0.5 * x * y * y)
    return y


def _dot_splat(a, b):
    s = a[0] * b[0]
    for c in range(1, NV):
        s = s + a[c] * b[c]
    return _splat(jnp.sum(s))


@functools.partial(
    pl.kernel,
    out_type=[
        jax.ShapeDtypeStruct((B,), jnp.float32),
        jax.ShapeDtypeStruct((B,), jnp.float32),
        jax.ShapeDtypeStruct((NW * LN,), jnp.int32),
        jax.ShapeDtypeStruct((NW * LN,), jnp.float32),
    ],
    mesh=_mesh,
    compiler_params=_params,
    scratch_types=[
        pltpu.VMEM((CB,), jnp.int32),
        pltpu.VMEM((CB,), jnp.int32),
        pltpu.VMEM((CB,), jnp.int32),
        pltpu.VMEM((CB,), jnp.int32),
        pltpu.VMEM((CB,), jnp.int32),
        pltpu.VMEM((CB,), jnp.int32),
        pltpu.VMEM((CB,), jnp.int32),
        pltpu.VMEM((NG, LN), jnp.int32),
        pltpu.VMEM((NG, LN), jnp.int32),
        pltpu.VMEM((NG, LN), jnp.int32),
        pltpu.VMEM((2, KB // 2, 2 * D), jnp.float32),
        pltpu.VMEM((2, KB // 2, 2 * D), jnp.float32),
        pltpu.VMEM((2, KB, 2 * D), jnp.float32),
        pltpu.VMEM((2, KB, 2 * D), jnp.float32),
        pltpu.VMEM((2, KB, 2 * D), jnp.float32),
        pltpu.VMEM((2, KB, 2 * D), jnp.float32),
        pltpu.VMEM((CB,), jnp.float32),
        pltpu.VMEM((CB,), jnp.float32),
        pltpu.VMEM((CB,), jnp.float32),
        pltpu.VMEM((LN,), jnp.int32),
        pltpu.VMEM((LN,), jnp.float32),
        pltpu.SemaphoreType.DMA,
        pltpu.SemaphoreType.DMA,
    ],
)
def _phase_a(emb_h2, emb_t2, bh, bt, br, mk, rel2, ent2, rel_t2,
             pos_hbm, neg_hbm, si_hbm, sf_hbm,
             idxh_v, idxt_v, idxr_v, mask_v, ih2_v, it2_v, ir2_v,
             ph_v, pt_v, pr_v, eh_v, et_v, ht_v, tt_v, rr_v, rt_v,
             score_v, pos_v, neg_v, si_v, sf_v, sem0, sem1):
    w = lax.axis_index("s") * NC + lax.axis_index("c")
    base = w * CB
    pltpu.sync_copy(bh.at[pl.ds(base, CB)], idxh_v)
    pltpu.sync_copy(bt.at[pl.ds(base, CB)], idxt_v)
    pltpu.sync_copy(br.at[pl.ds(base, CB)], idxr_v)
    pltpu.sync_copy(mk.at[pl.ds(base, CB)], mask_v)

    iota = lax.iota(jnp.int32, LN)
    one = jnp.ones((LN,), jnp.int32)
    s64 = jnp.full((LN,), D, jnp.int32)
    for g in range(NG):
        tgt = _splat(g * LN) + iota
        for src_v, dst_v, par_v in ((idxh_v, ih2_v, ph_v),
                                    (idxt_v, it2_v, pt_v),
                                    (idxr_v, ir2_v, pr_v)):
            ix = src_v[pl.ds(g * LN, LN)]
            plsc.store_scatter(dst_v, [tgt], ix >> 1)
            par_v[g, pl.ds(0, LN)] = (ix & one) * s64

    sems = (sem0, sem1)
    NB = CB // KB
    GPB = KB // LN

    def copies(blk, p):
        o = blk * KB
        eo = pl.multiple_of((base + o) // 2, 32)
        return [
            (ent2.at[ih2_v.at[pl.ds(o, KB)]], ht_v.at[p]),
            (ent2.at[it2_v.at[pl.ds(o, KB)]], tt_v.at[p]),
            (rel2.at[ir2_v.at[pl.ds(o, KB)]], rr_v.at[p]),
            (rel_t2.at[ir2_v.at[pl.ds(o, KB)]], rt_v.at[p]),
            (emb_h2.at[pl.ds(eo, KB // 2)], eh_v.at[p]),
            (emb_t2.at[pl.ds(eo, KB // 2)], et_v.at[p]),
        ]

    for src, dst in copies(0, 0):
        pltpu.async_copy(src, dst, sem0)

    def body(g, carry):
        blk = g // GPB
        p = blk % 2
        first = (g % GPB) == 0
        for pi in (0, 1):
            @pl.when(first & (p == pi))
            def _(pi=pi, blk=blk):
                for src, dst in copies(blk, pi):
                    pltpu.make_async_copy(src, dst, sems[pi]).wait()

                @pl.when(blk < NB - 1)
                def _(pi=pi, blk=blk):
                    for src, dst in copies(blk + 1, 1 - pi):
                        pltpu.async_copy(src, dst, sems[1 - pi])

        gl = g % GPB
        phv = ph_v[g, pl.ds(0, LN)]
        ptv = pt_v[g, pl.ds(0, LN)]
        prv = pr_v[g, pl.ds(0, LN)]
        svec = jnp.zeros((LN,), jnp.float32)
        for k in range(LN):
            i = gl * LN + k
            mh = _splat(phv[k]) == 0
            mt = _splat(ptv[k]) == 0
            mr = _splat(prv[k]) == 0
            ei = gl * (LN // 2) + k // 2
            eoff = D * (k % 2)
            eh = [eh_v[p, ei, pl.ds(eoff + LN * c, LN)] for c in range(NV)]
            et = [et_v[p, ei, pl.ds(eoff + LN * c, LN)] for c in range(NV)]

            def half(ref, row, m, c):
                lo = ref[p, row, pl.ds(LN * c, LN)]
                hi = ref[p, row, pl.ds(D + LN * c, LN)]
                return jnp.where(m, lo, hi)

            ht = [half(ht_v, i, mh, c) for c in range(NV)]
            tt = [half(tt_v, i, mt, c) for c in range(NV)]
            rr = [half(rr_v, i, mr, c) for c in range(NV)]
            rt = [half(rt_v, i, mr, c) for c in range(NV)]
            dh = _dot_splat(eh, ht)
            dt = _dot_splat(et, tt)
            h = [eh[c] + dh * rt[c] for c in range(NV)]
            t = [et[c] + dt * rt[c] for c in range(NV)]
            ih = _rsqrt(_dot_splat(h, h))
            it = _rsqrt(_dot_splat(t, t))
            ir = _rsqrt(_dot_splat(rr, rr))
            acc = jnp.abs(h[0] * ih + rr[0] * ir - t[0] * it)
            for c in range(1, NV):
                acc = acc + jnp.abs(h[c] * ih + rr[c] * ir - t[c] * it)
            s = jnp.float32(MARGIN) - jnp.sum(acc)
            svec = jnp.where(iota == k, _splat(s), svec)
        plsc.store_scatter(score_v, [_splat(g * LN) + iota], svec)
        return carry

    lax.fori_loop(0, NG, body, jnp.int32(0))

    rank_c = jnp.int32(0)
    accp = jnp.zeros((LN,), jnp.float32)
    accn = jnp.zeros((LN,), jnp.float32)
    for j in range(NG):
        m = mask_v[pl.ds(LN * j, LN)]
        s = score_v[pl.ds(LN * j, LN)]
        cum = plsc.cumsum(m)
        rank_p = _splat(rank_c) + cum - m
        mb = m > 0
        plsc.store_scatter(pos_v, [rank_p], s, mask=mb)
        rank_n = (jnp.int32(LN * j) + iota) - rank_p
        plsc.store_scatter(neg_v, [rank_n], s, mask=jnp.logical_not(mb))
        rank_c = rank_c + cum[LN - 1]
        accp = accp + jnp.where(mb, s, 0.0)
        accn = accn + jnp.where(mb, 0.0, s)

    zi = jnp.zeros((LN,), jnp.int32)
    zf = jnp.zeros((LN,), jnp.float32)
    si_v[...] = jnp.where(iota == 0, _splat(rank_c), zi)
    sf = jnp.where(iota == 0, _splat(jnp.sum(accp)), zf)
    sf_v[...] = jnp.where(iota == 1, _splat(jnp.sum(accn)), sf)
    pltpu.sync_copy(pos_v, pos_hbm.at[pl.ds(base, CB)])
    pltpu.sync_copy(neg_v, neg_hbm.at[pl.ds(base, CB)])
    pltpu.sync_copy(si_v, si_hbm.at[pl.ds(w * LN, LN)])
    pltpu.sync_copy(sf_v, sf_hbm.at[pl.ds(w * LN, LN)])


@functools.partial(
    pl.kernel,
    out_type=jax.ShapeDtypeStruct((NW * LN,), jnp.float32),
    mesh=_mesh,
    compiler_params=_params,
    scratch_types=[
        pltpu.VMEM((B,), jnp.float32),
        pltpu.VMEM((B,), jnp.float32),
        pltpu.VMEM((NW * LN,), jnp.int32),
        pltpu.VMEM((NW * LN,), jnp.float32),
        pltpu.SMEM((NW,), jnp.int32),
        pltpu.SMEM((NW,), jnp.int32),
        pltpu.VMEM((LN,), jnp.float32),
    ],
)
def _phase_b(pos_hbm, neg_hbm, si_hbm, sf_hbm, part_hbm,
             pos_v, neg_v, si_v, sf_v, offp_v, offn_v, stage_v):
    w = lax.axis_index("s") * NC + lax.axis_index("c")
    pltpu.sync_copy(pos_hbm, pos_v)
    pltpu.sync_copy(neg_hbm, neg_v)
    pltpu.sync_copy(si_hbm, si_v)
    pltpu.sync_copy(sf_hbm, sf_v)

    op = jnp.int32(0)
    on = jnp.int32(0)
    accf = sf_v[pl.ds(0, LN)]
    for j in range(NW):
        offp_v[j] = op
        offn_v[j] = on
        cj = si_v[pl.ds(j * LN, LN)][0]
        op = op + cj
        on = on + (jnp.int32(CB) - cj)
        if j > 0:
            accf = accf + sf_v[pl.ds(j * LN, LN)]

    Pv = _splat(op)
    Nv = jnp.int32(B) - Pv
    Lv = jnp.maximum(Pv, Nv)
    mean_p = _splat(accf[0]) / Pv.astype(jnp.float32)
    mean_n = _splat(accf[1]) / Nv.astype(jnp.float32)
    iota = lax.iota(jnp.int32, LN)
    negM = jnp.full((LN,), -MARGIN, jnp.float32)

    def body(v, acc):
        ranks = _splat(w * CB + v * LN) + iota
        selr_p = jnp.zeros((LN,), jnp.int32)
        selo_p = jnp.zeros((LN,), jnp.int32)
        selr_n = jnp.zeros((LN,), jnp.int32)
        selo_n = jnp.zeros((LN,), jnp.int32)
        for j in range(NW):
            oj = _splat(offp_v[j])
            le = oj <= ranks
            selr_p = jnp.where(le, j, selr_p)
            selo_p = jnp.where(le, oj, selo_p)
            oj = _splat(offn_v[j])
            le = oj <= ranks
            selr_n = jnp.where(le, j, selr_n)
            selo_n = jnp.where(le, oj, selo_n)
        colp = jnp.minimum(ranks - selo_p, CB - 1)
        coln = jnp.minimum(ranks - selo_n, CB - 1)
        pv = plsc.load_gather(pos_v, [selr_p * CB + colp])
        nv = plsc.load_gather(neg_v, [selr_n * CB + coln])
        p = jnp.where(ranks < Pv, pv, mean_p)
        n = jnp.where(ranks < Nv, nv, mean_n)
        term = jnp.maximum(p - n, negM)
        return acc + jnp.where(ranks < Lv, term, 0.0)

    acc = lax.fori_loop(0, CB // LN, body, jnp.zeros((LN,), jnp.float32))
    stage_v[...] = acc
    pltpu.sync_copy(stage_v, part_hbm.at[pl.ds(w * LN, LN)])


@functools.partial(
    pl.kernel,
    out_type=jax.ShapeDtypeStruct((LN,), jnp.float32),
    mesh=_mesh,
    compiler_params=_params,
    scratch_types=[
        pltpu.VMEM((NW * LN,), jnp.float32),
        pltpu.VMEM((NW * LN,), jnp.int32),
        pltpu.VMEM((LN,), jnp.float32),
    ],
)
def _phase_c(part_hbm, si_hbm, out_hbm, part_v, si_v, stage_v):
    w = lax.axis_index("s") * NC + lax.axis_index("c")
    pltpu.sync_copy(part_hbm, part_v)
    pltpu.sync_copy(si_hbm, si_v)
    acc = part_v[pl.ds(0, LN)]
    P = si_v[pl.ds(0, LN)][0]
    for j in range(1, NW):
        acc = acc + part_v[pl.ds(j * LN, LN)]
        P = P + si_v[pl.ds(j * LN, LN)][0]
    total = _splat(jnp.sum(acc))
    Pv = _splat(P)
    Lv = jnp.maximum(Pv, jnp.int32(B) - Pv).astype(jnp.float32)
    stage_v[...] = total / Lv + jnp.float32(MARGIN)

    @pl.when(w == 0)
    def _():
        pltpu.sync_copy(stage_v, out_hbm)


def _make_repack(M):
    NCOL = (M + 127) // 128
    KPW = (NCOL + NW - 1) // NW
    LASTC = NCOL - 1

    @functools.partial(
        pl.kernel,
        out_type=jax.ShapeDtypeStruct((64 * NCOL, 2 * D), jnp.float32),
        mesh=_mesh,
        compiler_params=_params,
        scratch_types=[
            pltpu.VMEM((2, D, 128), jnp.float32),
            pltpu.VMEM((2, 64, 2 * D), jnp.float32),
            pltpu.SemaphoreType.DMA,
            pltpu.SemaphoreType.DMA,
            pltpu.SemaphoreType.DMA,
        ],
    )
    def rp(tin, tout, in_v, out_v, semA, semB, semW):
        w = lax.axis_index("s") * NC + lax.axis_index("c")
        sems = (semA, semB)
        rows_cb = [lax.iota(jnp.int32, LN) + LN * cb for cb in range(NV)]

        def in_src(c):
            return tin.at[:, pl.ds(pl.multiple_of(c * 128, 128), 128)]

        def out_dst(c):
            return tout.at[pl.ds(pl.multiple_of(c * 64, 64), 64)]

        def body(k, carry):
            c = w + NW * k

            @pl.when(c < NCOL)
            def _():
                pltpu.sync_copy(in_src(c), in_v.at[0])
                for el in range(128):
                    col = _splat(jnp.int32(el))
                    for cb in range(NV):
                        v = plsc.load_gather(in_v.at[0], [rows_cb[cb], col])
                        out_v[0, el // 2, pl.ds(D * (el % 2) + LN * cb, LN)] = v
                pltpu.sync_copy(out_v.at[0], out_dst(c))
            return carry

        lax.fori_loop(0, KPW, body, jnp.int32(0))

    return rp


def kernel(emb_h, emb_t, batch_h, batch_t, batch_r, mask,
           rel_embeddings, ent_transfer, rel_transfer):
    bh = batch_h.astype(jnp.int32)
    bt = batch_t.astype(jnp.int32)
    br = batch_r.astype(jnp.int32)
    mk = mask.astype(jnp.int32)
    ent2 = _make_repack(ent_transfer.shape[0])(ent_transfer.T)
    emb_h2 = _make_repack(B)(emb_h.T)
    emb_t2 = _make_repack(B)(emb_t.T)
    rel2 = rel_embeddings.reshape(rel_embeddings.shape[0] // 2, 2 * D)
    rel_t2 = rel_transfer.reshape(rel_transfer.shape[0] // 2, 2 * D)
    pos, neg, si, sf = _phase_a(emb_h2, emb_t2, bh, bt, br, mk,
                                rel2, ent2, rel_t2)
    part = _phase_b(pos, neg, si, sf)
    out16 = _phase_c(part, si)
    return out16[0]


# speedup vs baseline: 1.2181x; 1.2181x over previous
import functools

import jax
import jax.numpy as jnp
from jax import lax
from jax.experimental import pallas as pl
from jax.experimental.pallas import tpu as pltpu
from jax.experimental.pallas import tpu_sc as plsc

B = 16384
D = 64
MARGIN = 4.0
NC = 2
NS = 16
NW = NC * NS
CB = B // NW
KB = 64
LN = 16
NV = 4
NG = CB // LN

_mesh = plsc.VectorSubcoreMesh(
    core_axis_name="c", subcore_axis_name="s", num_cores=NC, num_subcores=NS
)
_params = pltpu.CompilerParams(needs_layout_passes=False)


def _splat(x, dtype=None):
    v = jnp.broadcast_to(x, (LN,))
    return v if dtype is None else v.astype(dtype)


def _rsqrt(x):
    i = plsc.bitcast(x, jnp.int32)
    y = plsc.bitcast(jnp.int32(0x5F3759DF) - (i >> 1), jnp.float32)
    for _ in range(3):
        y = y * (1.5 - 0.5 * x * y * y)
    return y


def _dot_splat(a, b):
    s = a[0] * b[0]
    for c in range(1, NV):
        s = s + a[c] * b[c]
    return _splat(jnp.sum(s))


@functools.partial(
    pl.kernel,
    out_type=[
        jax.ShapeDtypeStruct((B,), jnp.float32),
        jax.ShapeDtypeStruct((B,), jnp.float32),
        jax.ShapeDtypeStruct((NW * LN,), jnp.int32),
        jax.ShapeDtypeStruct((NW * LN,), jnp.float32),
    ],
    mesh=_mesh,
    compiler_params=_params,
    scratch_types=[
        pltpu.VMEM((CB,), jnp.int32),
        pltpu.VMEM((CB,), jnp.int32),
        pltpu.VMEM((CB,), jnp.int32),
        pltpu.VMEM((CB,), jnp.int32),
        pltpu.VMEM((CB,), jnp.int32),
        pltpu.VMEM((CB,), jnp.int32),
        pltpu.VMEM((CB,), jnp.int32),
        pltpu.VMEM((NG, LN), jnp.int32),
        pltpu.VMEM((NG, LN), jnp.int32),
        pltpu.VMEM((NG, LN), jnp.int32),
        pltpu.VMEM((2, KB // 2, 2 * D), jnp.float32),
        pltpu.VMEM((2, KB // 2, 2 * D), jnp.float32),
        pltpu.VMEM((2, KB, 2 * D), jnp.float32),
        pltpu.VMEM((2, KB, 2 * D), jnp.float32),
        pltpu.VMEM((2, KB, 2 * D), jnp.float32),
        pltpu.VMEM((2, KB, 2 * D), jnp.float32),
        pltpu.VMEM((CB,), jnp.float32),
        pltpu.VMEM((CB,), jnp.float32),
        pltpu.VMEM((CB,), jnp.float32),
        pltpu.VMEM((LN,), jnp.int32),
        pltpu.VMEM((LN,), jnp.float32),
        pltpu.SemaphoreType.DMA,
        pltpu.SemaphoreType.DMA,
    ],
)
def _phase_a(emb_h2, emb_t2, bh, bt, br, mk, rel2, ent2, rel_t2,
             pos_hbm, neg_hbm, si_hbm, sf_hbm,
             idxh_v, idxt_v, idxr_v, mask_v, ih2_v, it2_v, ir2_v,
             ph_v, pt_v, pr_v, eh_v, et_v, ht_v, tt_v, rr_v, rt_v,
             score_v, pos_v, neg_v, si_v, sf_v, sem0, sem1):
    w = lax.axis_index("s") * NC + lax.axis_index("c")
    base = w * CB
    pltpu.sync_copy(bh.at[pl.ds(base, CB)], idxh_v)
    pltpu.sync_copy(bt.at[pl.ds(base, CB)], idxt_v)
    pltpu.sync_copy(br.at[pl.ds(base, CB)], idxr_v)
    pltpu.sync_copy(mk.at[pl.ds(base, CB)], mask_v)

    iota = lax.iota(jnp.int32, LN)
    one = jnp.ones((LN,), jnp.int32)
    s64 = jnp.full((LN,), D, jnp.int32)
    for g in range(NG):
        tgt = _splat(g * LN) + iota
        for src_v, dst_v, par_v in ((idxh_v, ih2_v, ph_v),
                                    (idxt_v, it2_v, pt_v),
                                    (idxr_v, ir2_v, pr_v)):
            ix = src_v[pl.ds(g * LN, LN)]
            plsc.store_scatter(dst_v, [tgt], ix >> 1)
            par_v[g, pl.ds(0, LN)] = (ix & one) * s64

    sems = (sem0, sem1)
    NB = CB // KB
    GPB = KB // LN

    def copies(blk, p):
        o = blk * KB
        eo = pl.multiple_of((base + o) // 2, 32)
        return [
            (ent2.at[ih2_v.at[pl.ds(o, KB)]], ht_v.at[p]),
            (ent2.at[it2_v.at[pl.ds(o, KB)]], tt_v.at[p]),
            (rel2.at[ir2_v.at[pl.ds(o, KB)]], rr_v.at[p]),
            (rel_t2.at[ir2_v.at[pl.ds(o, KB)]], rt_v.at[p]),
            (emb_h2.at[pl.ds(eo, KB // 2)], eh_v.at[p]),
            (emb_t2.at[pl.ds(eo, KB // 2)], et_v.at[p]),
        ]

    for src, dst in copies(0, 0):
        pltpu.async_copy(src, dst, sem0)

    def body(g, carry):
        blk = g // GPB
        p = blk % 2
        first = (g % GPB) == 0
        for pi in (0, 1):
            @pl.when(first & (p == pi))
            def _(pi=pi, blk=blk):
                for src, dst in copies(blk, pi):
                    pltpu.make_async_copy(src, dst, sems[pi]).wait()

                @pl.when(blk < NB - 1)
                def _(pi=pi, blk=blk):
                    for src, dst in copies(blk + 1, 1 - pi):
                        pltpu.async_copy(src, dst, sems[1 - pi])

        gl = g % GPB
        phv = ph_v[g, pl.ds(0, LN)]
        ptv = pt_v[g, pl.ds(0, LN)]
        prv = pr_v[g, pl.ds(0, LN)]
        svec = jnp.zeros((LN,), jnp.float32)
        for k in range(LN):
            i = gl * LN + k
            mh = _splat(phv[k]) == 0
            mt = _splat(ptv[k]) == 0
            mr = _splat(prv[k]) == 0
            ei = gl * (LN // 2) + k // 2
            eoff = D * (k % 2)
            eh = [eh_v[p, ei, pl.ds(eoff + LN * c, LN)] for c in range(NV)]
            et = [et_v[p, ei, pl.ds(eoff + LN * c, LN)] for c in range(NV)]

            def half(ref, row, m, c):
                lo = ref[p, row, pl.ds(LN * c, LN)]
                hi = ref[p, row, pl.ds(D + LN * c, LN)]
                return jnp.where(m, lo, hi)

            ht = [half(ht_v, i, mh, c) for c in range(NV)]
            tt = [half(tt_v, i, mt, c) for c in range(NV)]
            rr = [half(rr_v, i, mr, c) for c in range(NV)]
            rt = [half(rt_v, i, mr, c) for c in range(NV)]
            dh = _dot_splat(eh, ht)
            dt = _dot_splat(et, tt)
            h = [eh[c] + dh * rt[c] for c in range(NV)]
            t = [et[c] + dt * rt[c] for c in range(NV)]
            ih = _rsqrt(_dot_splat(h, h))
            it = _rsqrt(_dot_splat(t, t))
            ir = _rsqrt(_dot_splat(rr, rr))
            acc = jnp.abs(h[0] * ih + rr[0] * ir - t[0] * it)
            for c in range(1, NV):
                acc = acc + jnp.abs(h[c] * ih + rr[c] * ir - t[c] * it)
            s = jnp.float32(MARGIN) - jnp.sum(acc)
            svec = jnp.where(iota == k, _splat(s), svec)
        plsc.store_scatter(score_v, [_splat(g * LN) + iota], svec)
        return carry

    lax.fori_loop(0, NG, body, jnp.int32(0))

    rank_c = jnp.int32(0)
    accp = jnp.zeros((LN,), jnp.float32)
    accn = jnp.zeros((LN,), jnp.float32)
    for j in range(NG):
        m = mask_v[pl.ds(LN * j, LN)]
        s = score_v[pl.ds(LN * j, LN)]
        cum = plsc.cumsum(m)
        rank_p = _splat(rank_c) + cum - m
        mb = m > 0
        plsc.store_scatter(pos_v, [rank_p], s, mask=mb)
        rank_n = (jnp.int32(LN * j) + iota) - rank_p
        plsc.store_scatter(neg_v, [rank_n], s, mask=jnp.logical_not(mb))
        rank_c = rank_c + cum[LN - 1]
        accp = accp + jnp.where(mb, s, 0.0)
        accn = accn + jnp.where(mb, 0.0, s)

    zi = jnp.zeros((LN,), jnp.int32)
    zf = jnp.zeros((LN,), jnp.float32)
    si_v[...] = jnp.where(iota == 0, _splat(rank_c), zi)
    sf = jnp.where(iota == 0, _splat(jnp.sum(accp)), zf)
    sf_v[...] = jnp.where(iota == 1, _splat(jnp.sum(accn)), sf)
    pltpu.sync_copy(pos_v, pos_hbm.at[pl.ds(base, CB)])
    pltpu.sync_copy(neg_v, neg_hbm.at[pl.ds(base, CB)])
    pltpu.sync_copy(si_v, si_hbm.at[pl.ds(w * LN, LN)])
    pltpu.sync_copy(sf_v, sf_hbm.at[pl.ds(w * LN, LN)])


@functools.partial(
    pl.kernel,
    out_type=jax.ShapeDtypeStruct((NW * LN,), jnp.float32),
    mesh=_mesh,
    compiler_params=_params,
    scratch_types=[
        pltpu.VMEM((B,), jnp.float32),
        pltpu.VMEM((B,), jnp.float32),
        pltpu.VMEM((NW * LN,), jnp.int32),
        pltpu.VMEM((NW * LN,), jnp.float32),
        pltpu.SMEM((NW,), jnp.int32),
        pltpu.SMEM((NW,), jnp.int32),
        pltpu.VMEM((LN,), jnp.float32),
    ],
)
def _phase_b(pos_hbm, neg_hbm, si_hbm, sf_hbm, part_hbm,
             pos_v, neg_v, si_v, sf_v, offp_v, offn_v, stage_v):
    w = lax.axis_index("s") * NC + lax.axis_index("c")
    pltpu.sync_copy(pos_hbm, pos_v)
    pltpu.sync_copy(neg_hbm, neg_v)
    pltpu.sync_copy(si_hbm, si_v)
    pltpu.sync_copy(sf_hbm, sf_v)

    op = jnp.int32(0)
    on = jnp.int32(0)
    accf = sf_v[pl.ds(0, LN)]
    for j in range(NW):
        offp_v[j] = op
        offn_v[j] = on
        cj = si_v[pl.ds(j * LN, LN)][0]
        op = op + cj
        on = on + (jnp.int32(CB) - cj)
        if j > 0:
            accf = accf + sf_v[pl.ds(j * LN, LN)]

    Pv = _splat(op)
    Nv = jnp.int32(B) - Pv
    Lv = jnp.maximum(Pv, Nv)
    mean_p = _splat(accf[0]) / Pv.astype(jnp.float32)
    mean_n = _splat(accf[1]) / Nv.astype(jnp.float32)
    iota = lax.iota(jnp.int32, LN)
    negM = jnp.full((LN,), -MARGIN, jnp.float32)

    def body(v, acc):
        ranks = _splat(w * CB + v * LN) + iota
        selr_p = jnp.zeros((LN,), jnp.int32)
        selo_p = jnp.zeros((LN,), jnp.int32)
        selr_n = jnp.zeros((LN,), jnp.int32)
        selo_n = jnp.zeros((LN,), jnp.int32)
        for j in range(NW):
            oj = _splat(offp_v[j])
            le = oj <= ranks
            selr_p = jnp.where(le, j, selr_p)
            selo_p = jnp.where(le, oj, selo_p)
            oj = _splat(offn_v[j])
            le = oj <= ranks
            selr_n = jnp.where(le, j, selr_n)
            selo_n = jnp.where(le, oj, selo_n)
        colp = jnp.minimum(ranks - selo_p, CB - 1)
        coln = jnp.minimum(ranks - selo_n, CB - 1)
        pv = plsc.load_gather(pos_v, [selr_p * CB + colp])
        nv = plsc.load_gather(neg_v, [selr_n * CB + coln])
        p = jnp.where(ranks < Pv, pv, mean_p)
        n = jnp.where(ranks < Nv, nv, mean_n)
        term = jnp.maximum(p - n, negM)
        return acc + jnp.where(ranks < Lv, term, 0.0)

    acc = lax.fori_loop(0, CB // LN, body, jnp.zeros((LN,), jnp.float32))
    stage_v[...] = acc
    pltpu.sync_copy(stage_v, part_hbm.at[pl.ds(w * LN, LN)])


@functools.partial(
    pl.kernel,
    out_type=jax.ShapeDtypeStruct((LN,), jnp.float32),
    mesh=_mesh,
    compiler_params=_params,
    scratch_types=[
        pltpu.VMEM((NW * LN,), jnp.float32),
        pltpu.VMEM((NW * LN,), jnp.int32),
        pltpu.VMEM((LN,), jnp.float32),
    ],
)
def _phase_c(part_hbm, si_hbm, out_hbm, part_v, si_v, stage_v):
    w = lax.axis_index("s") * NC + lax.axis_index("c")
    pltpu.sync_copy(part_hbm, part_v)
    pltpu.sync_copy(si_hbm, si_v)
    acc = part_v[pl.ds(0, LN)]
    P = si_v[pl.ds(0, LN)][0]
    for j in range(1, NW):
        acc = acc + part_v[pl.ds(j * LN, LN)]
        P = P + si_v[pl.ds(j * LN, LN)][0]
    total = _splat(jnp.sum(acc))
    Pv = _splat(P)
    Lv = jnp.maximum(Pv, jnp.int32(B) - Pv).astype(jnp.float32)
    stage_v[...] = total / Lv + jnp.float32(MARGIN)

    @pl.when(w == 0)
    def _():
        pltpu.sync_copy(stage_v, out_hbm)


def _make_repack(M):
    NCOL = (M + 127) // 128
    KPW = (NCOL + NW - 1) // NW
    LASTC = NCOL - 1

    @functools.partial(
        pl.kernel,
        out_type=jax.ShapeDtypeStruct((64 * NCOL, 2 * D), jnp.float32),
        mesh=_mesh,
        compiler_params=_params,
        scratch_types=[
            pltpu.VMEM((2, D, 128), jnp.float32),
            pltpu.VMEM((2, 64, 2 * D), jnp.float32),
            pltpu.SemaphoreType.DMA,
            pltpu.SemaphoreType.DMA,
            pltpu.SemaphoreType.DMA,
        ],
    )
    def rp(tin, tout, in_v, out_v, semA, semB, semW):
        w = lax.axis_index("s") * NC + lax.axis_index("c")
        sems = (semA, semB)
        rows_cb = [lax.iota(jnp.int32, LN) + LN * cb for cb in range(NV)]

        def in_src(c):
            return tin.at[:, pl.ds(pl.multiple_of(c * 128, 128), 128)]

        def out_dst(c):
            return tout.at[pl.ds(pl.multiple_of(c * 64, 64), 64)]

        pltpu.async_copy(in_src(w), in_v.at[0], semA)

        def body(k, carry):
            c = w + NW * k
            p = k % 2
            for pi in (0, 1):
                on_p = p == pi

                @pl.when(on_p & (c + NW < NCOL) & (k + 1 < KPW))
                def _(pi=pi):
                    pltpu.async_copy(in_src(c + NW), in_v.at[1 - pi], sems[1 - pi])

                @pl.when(on_p & (c < NCOL))
                def _(pi=pi):
                    pltpu.make_async_copy(in_src(c), in_v.at[pi], sems[pi]).wait()

                @pl.when(on_p & (k >= 2) & (c - 2 * NW < NCOL))
                def _(pi=pi):
                    pltpu.make_async_copy(out_v.at[pi], out_dst(c - 2 * NW), semW).wait()

            @pl.when(c < NCOL)
            def _():
                for el in range(128):
                    col = _splat(jnp.int32(el))
                    for cb in range(NV):
                        v = plsc.load_gather(in_v.at[p], [rows_cb[cb], col])
                        out_v[p, el // 2, pl.ds(D * (el % 2) + LN * cb, LN)] = v
                pltpu.async_copy(out_v.at[p], out_dst(c), semW)
            return carry

        lax.fori_loop(0, KPW, body, jnp.int32(0))
        for tk in (KPW - 2, KPW - 1):
            if tk >= 0:
                ct = w + NW * tk

                @pl.when(ct < NCOL)
                def _(tk=tk, ct=ct):
                    pltpu.make_async_copy(out_v.at[tk % 2], out_dst(ct), semW).wait()

    return rp


def kernel(emb_h, emb_t, batch_h, batch_t, batch_r, mask,
           rel_embeddings, ent_transfer, rel_transfer):
    bh = batch_h.astype(jnp.int32)
    bt = batch_t.astype(jnp.int32)
    br = batch_r.astype(jnp.int32)
    mk = mask.astype(jnp.int32)
    ent2 = _make_repack(ent_transfer.shape[0])(ent_transfer.T)
    emb_h2 = _make_repack(B)(emb_h.T)
    emb_t2 = _make_repack(B)(emb_t.T)
    rel2 = rel_embeddings.reshape(rel_embeddings.shape[0] // 2, 2 * D)
    rel_t2 = rel_transfer.reshape(rel_transfer.shape[0] // 2, 2 * D)
    pos, neg, si, sf = _phase_a(emb_h2, emb_t2, bh, bt, br, mk,
                                rel2, ent2, rel_t2)
    part = _phase_b(pos, neg, si, sf)
    out16 = _phase_c(part, si)
    return out16[0]


# speedup vs baseline: 4.6195x; 3.7925x over previous
import functools

import jax
import jax.numpy as jnp
from jax import lax
from jax.experimental import pallas as pl
from jax.experimental.pallas import tpu as pltpu
from jax.experimental.pallas import tpu_sc as plsc

B = 16384
D = 64
MARGIN = 4.0
NC = 2
NS = 16
NW = NC * NS
CB = B // NW
KB = 64
LN = 16
NV = 4
NG = CB // LN

_mesh = plsc.VectorSubcoreMesh(
    core_axis_name="c", subcore_axis_name="s", num_cores=NC, num_subcores=NS
)
_params = pltpu.CompilerParams(needs_layout_passes=False)


def _splat(x, dtype=None):
    v = jnp.broadcast_to(x, (LN,))
    return v if dtype is None else v.astype(dtype)


def _rsqrt(x):
    i = plsc.bitcast(x, jnp.int32)
    y = plsc.bitcast(jnp.int32(0x5F3759DF) - (i >> 1), jnp.float32)
    for _ in range(3):
        y = y * (1.5 - 0.5 * x * y * y)
    return y


def _dot_splat(a, b):
    s = a[0] * b[0]
    for c in range(1, NV):
        s = s + a[c] * b[c]
    return _splat(jnp.sum(s))


@functools.partial(
    pl.kernel,
    out_type=[
        jax.ShapeDtypeStruct((B,), jnp.float32),
        jax.ShapeDtypeStruct((B,), jnp.float32),
        jax.ShapeDtypeStruct((NW * LN,), jnp.int32),
        jax.ShapeDtypeStruct((NW * LN,), jnp.float32),
    ],
    mesh=_mesh,
    compiler_params=_params,
    scratch_types=[
        pltpu.VMEM((CB,), jnp.int32),
        pltpu.VMEM((CB,), jnp.int32),
        pltpu.VMEM((CB,), jnp.int32),
        pltpu.VMEM((CB,), jnp.int32),
        pltpu.VMEM((2, KB, 2 * D), jnp.float32),
        pltpu.VMEM((2, KB, 2 * D), jnp.float32),
        pltpu.VMEM((2, KB, 2 * D), jnp.float32),
        pltpu.VMEM((2, KB, 2 * D), jnp.float32),
        pltpu.VMEM((2, KB, 2 * D), jnp.float32),
        pltpu.VMEM((2, KB, 2 * D), jnp.float32),
        pltpu.VMEM((CB,), jnp.float32),
        pltpu.VMEM((CB,), jnp.float32),
        pltpu.VMEM((CB,), jnp.float32),
        pltpu.VMEM((LN,), jnp.int32),
        pltpu.VMEM((LN,), jnp.float32),
        pltpu.SemaphoreType.DMA,
        pltpu.SemaphoreType.DMA,
    ],
)
def _phase_a(emb_h2, emb_t2, bh, bt, br, mk, rel2, ent2, rel_t2,
             pos_hbm, neg_hbm, si_hbm, sf_hbm,
             idxh_v, idxt_v, idxr_v, mask_v, eh_v, et_v, ht_v, tt_v, rr_v, rt_v,
             score_v, pos_v, neg_v, si_v, sf_v, sem0, sem1):
    w = lax.axis_index("s") * NC + lax.axis_index("c")
    base = w * CB
    pltpu.sync_copy(bh.at[pl.ds(base, CB)], idxh_v)
    pltpu.sync_copy(bt.at[pl.ds(base, CB)], idxt_v)
    pltpu.sync_copy(br.at[pl.ds(base, CB)], idxr_v)
    pltpu.sync_copy(mk.at[pl.ds(base, CB)], mask_v)

    iota = lax.iota(jnp.int32, LN)
    sems = (sem0, sem1)
    NB = CB // KB
    GPB = KB // LN

    def copies(blk, p):
        o = blk * KB
        eo = pl.multiple_of(base + o, KB)
        return [
            (ent2.at[idxh_v.at[pl.ds(o, KB)]], ht_v.at[p]),
            (ent2.at[idxt_v.at[pl.ds(o, KB)]], tt_v.at[p]),
            (rel2.at[idxr_v.at[pl.ds(o, KB)]], rr_v.at[p]),
            (rel_t2.at[idxr_v.at[pl.ds(o, KB)]], rt_v.at[p]),
            (emb_h2.at[pl.ds(eo, KB)], eh_v.at[p]),
            (emb_t2.at[pl.ds(eo, KB)], et_v.at[p]),
        ]

    for src, dst in copies(0, 0):
        pltpu.async_copy(src, dst, sem0)

    def body(g, carry):
        blk = g // GPB
        p = blk % 2
        first = (g % GPB) == 0
        for pi in (0, 1):
            @pl.when(first & (p == pi))
            def _(pi=pi, blk=blk):
                for src, dst in copies(blk, pi):
                    pltpu.make_async_copy(src, dst, sems[pi]).wait()

                @pl.when(blk < NB - 1)
                def _(pi=pi, blk=blk):
                    for src, dst in copies(blk + 1, 1 - pi):
                        pltpu.async_copy(src, dst, sems[1 - pi])

        gl = g % GPB
        svec = jnp.zeros((LN,), jnp.float32)
        for k in range(LN):
            i = gl * LN + k
            eh = [eh_v[p, i, pl.ds(LN * c, LN)] for c in range(NV)]
            et = [et_v[p, i, pl.ds(LN * c, LN)] for c in range(NV)]
            ht = [ht_v[p, i, pl.ds(LN * c, LN)] for c in range(NV)]
            tt = [tt_v[p, i, pl.ds(LN * c, LN)] for c in range(NV)]
            rr = [rr_v[p, i, pl.ds(LN * c, LN)] for c in range(NV)]
            rt = [rt_v[p, i, pl.ds(LN * c, LN)] for c in range(NV)]
            dh = _dot_splat(eh, ht)
            dt = _dot_splat(et, tt)
            h = [eh[c] + dh * rt[c] for c in range(NV)]
            t = [et[c] + dt * rt[c] for c in range(NV)]
            ih = _rsqrt(_dot_splat(h, h))
            it = _rsqrt(_dot_splat(t, t))
            ir = _rsqrt(_dot_splat(rr, rr))
            acc = jnp.abs(h[0] * ih + rr[0] * ir - t[0] * it)
            for c in range(1, NV):
                acc = acc + jnp.abs(h[c] * ih + rr[c] * ir - t[c] * it)
            s = jnp.float32(MARGIN) - jnp.sum(acc)
            svec = jnp.where(iota == k, _splat(s), svec)
        plsc.store_scatter(score_v, [_splat(g * LN) + iota], svec)
        return carry

    lax.fori_loop(0, NG, body, jnp.int32(0))

    rank_c = jnp.int32(0)
    accp = jnp.zeros((LN,), jnp.float32)
    accn = jnp.zeros((LN,), jnp.float32)
    for j in range(NG):
        m = mask_v[pl.ds(LN * j, LN)]
        s = score_v[pl.ds(LN * j, LN)]
        cum = plsc.cumsum(m)
        rank_p = _splat(rank_c) + cum - m
        mb = m > 0
        plsc.store_scatter(pos_v, [rank_p], s, mask=mb)
        rank_n = (jnp.int32(LN * j) + iota) - rank_p
        plsc.store_scatter(neg_v, [rank_n], s, mask=jnp.logical_not(mb))
        rank_c = rank_c + cum[LN - 1]
        accp = accp + jnp.where(mb, s, 0.0)
        accn = accn + jnp.where(mb, 0.0, s)

    zi = jnp.zeros((LN,), jnp.int32)
    zf = jnp.zeros((LN,), jnp.float32)
    si_v[...] = jnp.where(iota == 0, _splat(rank_c), zi)
    sf = jnp.where(iota == 0, _splat(jnp.sum(accp)), zf)
    sf_v[...] = jnp.where(iota == 1, _splat(jnp.sum(accn)), sf)
    pltpu.sync_copy(pos_v, pos_hbm.at[pl.ds(base, CB)])
    pltpu.sync_copy(neg_v, neg_hbm.at[pl.ds(base, CB)])
    pltpu.sync_copy(si_v, si_hbm.at[pl.ds(w * LN, LN)])
    pltpu.sync_copy(sf_v, sf_hbm.at[pl.ds(w * LN, LN)])


@functools.partial(
    pl.kernel,
    out_type=jax.ShapeDtypeStruct((NW * LN,), jnp.float32),
    mesh=_mesh,
    compiler_params=_params,
    scratch_types=[
        pltpu.VMEM((B,), jnp.float32),
        pltpu.VMEM((B,), jnp.float32),
        pltpu.VMEM((NW * LN,), jnp.int32),
        pltpu.VMEM((NW * LN,), jnp.float32),
        pltpu.SMEM((NW,), jnp.int32),
        pltpu.SMEM((NW,), jnp.int32),
        pltpu.VMEM((LN,), jnp.float32),
    ],
)
def _phase_b(pos_hbm, neg_hbm, si_hbm, sf_hbm, part_hbm,
             pos_v, neg_v, si_v, sf_v, offp_v, offn_v, stage_v):
    w = lax.axis_index("s") * NC + lax.axis_index("c")
    pltpu.sync_copy(pos_hbm, pos_v)
    pltpu.sync_copy(neg_hbm, neg_v)
    pltpu.sync_copy(si_hbm, si_v)
    pltpu.sync_copy(sf_hbm, sf_v)

    op = jnp.int32(0)
    on = jnp.int32(0)
    accf = sf_v[pl.ds(0, LN)]
    for j in range(NW):
        offp_v[j] = op
        offn_v[j] = on
        cj = si_v[pl.ds(j * LN, LN)][0]
        op = op + cj
        on = on + (jnp.int32(CB) - cj)
        if j > 0:
            accf = accf + sf_v[pl.ds(j * LN, LN)]

    Pv = _splat(op)
    Nv = jnp.int32(B) - Pv
    Lv = jnp.maximum(Pv, Nv)
    mean_p = _splat(accf[0]) / Pv.astype(jnp.float32)
    mean_n = _splat(accf[1]) / Nv.astype(jnp.float32)
    iota = lax.iota(jnp.int32, LN)
    negM = jnp.full((LN,), -MARGIN, jnp.float32)

    def body(v, acc):
        ranks = _splat(w * CB + v * LN) + iota
        selr_p = jnp.zeros((LN,), jnp.int32)
        selo_p = jnp.zeros((LN,), jnp.int32)
        selr_n = jnp.zeros((LN,), jnp.int32)
        selo_n = jnp.zeros((LN,), jnp.int32)
        for j in range(NW):
            oj = _splat(offp_v[j])
            le = oj <= ranks
            selr_p = jnp.where(le, j, selr_p)
            selo_p = jnp.where(le, oj, selo_p)
            oj = _splat(offn_v[j])
            le = oj <= ranks
            selr_n = jnp.where(le, j, selr_n)
            selo_n = jnp.where(le, oj, selo_n)
        colp = jnp.minimum(ranks - selo_p, CB - 1)
        coln = jnp.minimum(ranks - selo_n, CB - 1)
        pv = plsc.load_gather(pos_v, [selr_p * CB + colp])
        nv = plsc.load_gather(neg_v, [selr_n * CB + coln])
        p = jnp.where(ranks < Pv, pv, mean_p)
        n = jnp.where(ranks < Nv, nv, mean_n)
        term = jnp.maximum(p - n, negM)
        return acc + jnp.where(ranks < Lv, term, 0.0)

    acc = lax.fori_loop(0, CB // LN, body, jnp.zeros((LN,), jnp.float32))
    stage_v[...] = acc
    pltpu.sync_copy(stage_v, part_hbm.at[pl.ds(w * LN, LN)])


@functools.partial(
    pl.kernel,
    out_type=jax.ShapeDtypeStruct((LN,), jnp.float32),
    mesh=_mesh,
    compiler_params=_params,
    scratch_types=[
        pltpu.VMEM((NW * LN,), jnp.float32),
        pltpu.VMEM((NW * LN,), jnp.int32),
        pltpu.VMEM((LN,), jnp.float32),
    ],
)
def _phase_c(part_hbm, si_hbm, out_hbm, part_v, si_v, stage_v):
    w = lax.axis_index("s") * NC + lax.axis_index("c")
    pltpu.sync_copy(part_hbm, part_v)
    pltpu.sync_copy(si_hbm, si_v)
    acc = part_v[pl.ds(0, LN)]
    P = si_v[pl.ds(0, LN)][0]
    for j in range(1, NW):
        acc = acc + part_v[pl.ds(j * LN, LN)]
        P = P + si_v[pl.ds(j * LN, LN)][0]
    total = _splat(jnp.sum(acc))
    Pv = _splat(P)
    Lv = jnp.maximum(Pv, jnp.int32(B) - Pv).astype(jnp.float32)
    stage_v[...] = total / Lv + jnp.float32(MARGIN)

    @pl.when(w == 0)
    def _():
        pltpu.sync_copy(stage_v, out_hbm)


def _repack_body(in_ref, out_ref):
    x = in_ref[...]
    z = jnp.zeros((x.shape[1], D), jnp.float32)
    out_ref[...] = jnp.concatenate([x.T.astype(jnp.float32), z], axis=1)


def _repack(table_t, blk):
    m = table_t.shape[1]
    return pl.pallas_call(
        _repack_body,
        grid=(pl.cdiv(m, blk),),
        in_specs=[pl.BlockSpec((D, blk), lambda i: (0, i))],
        out_specs=pl.BlockSpec((blk, 2 * D), lambda i: (i, 0)),
        out_shape=jax.ShapeDtypeStruct((m, 2 * D), jnp.float32),
    )(table_t)


def kernel(emb_h, emb_t, batch_h, batch_t, batch_r, mask,
           rel_embeddings, ent_transfer, rel_transfer):
    bh = batch_h.astype(jnp.int32)
    bt = batch_t.astype(jnp.int32)
    br = batch_r.astype(jnp.int32)
    mk = mask.astype(jnp.int32)
    ent2 = _repack(ent_transfer.T, 4096)
    rel2 = _repack(rel_embeddings.T, 512)
    rel_t2 = _repack(rel_transfer.T, 512)
    emb_h2 = _repack(emb_h.T, 2048)
    emb_t2 = _repack(emb_t.T, 2048)
    pos, neg, si, sf = _phase_a(emb_h2, emb_t2, bh, bt, br, mk,
                                rel2, ent2, rel_t2)
    part = _phase_b(pos, neg, si, sf)
    out16 = _phase_c(part, si)
    return out16[0]


# speedup vs baseline: 5.9770x; 1.2939x over previous
import functools

import jax
import jax.numpy as jnp
from jax import lax
from jax.experimental import pallas as pl
from jax.experimental.pallas import tpu as pltpu
from jax.experimental.pallas import tpu_sc as plsc

B = 16384
D = 64
MARGIN = 4.0
NC = 2
NS = 16
NW = NC * NS
CB = B // NW
KB = 64
LN = 16
NV = 4
NG = CB // LN

_mesh = plsc.VectorSubcoreMesh(
    core_axis_name="c", subcore_axis_name="s", num_cores=NC, num_subcores=NS
)
_params = pltpu.CompilerParams(needs_layout_passes=False)


def _splat(x, dtype=None):
    v = jnp.broadcast_to(x, (LN,))
    return v if dtype is None else v.astype(dtype)


def _rsqrt(x):
    i = plsc.bitcast(x, jnp.int32)
    y = plsc.bitcast(jnp.int32(0x5F3759DF) - (i >> 1), jnp.float32)
    for _ in range(3):
        y = y * (1.5 - 0.5 * x * y * y)
    return y


def _dot_splat(a, b):
    s = a[0] * b[0]
    for c in range(1, NV):
        s = s + a[c] * b[c]
    return _splat(jnp.sum(s))


@functools.partial(
    pl.kernel,
    out_type=[
        jax.ShapeDtypeStruct((B,), jnp.float32),
        jax.ShapeDtypeStruct((B,), jnp.float32),
        jax.ShapeDtypeStruct((NW * LN,), jnp.int32),
        jax.ShapeDtypeStruct((NW * LN,), jnp.float32),
    ],
    mesh=_mesh,
    compiler_params=_params,
    scratch_types=[
        pltpu.VMEM((CB,), jnp.int32),
        pltpu.VMEM((CB,), jnp.int32),
        pltpu.VMEM((CB,), jnp.int32),
        pltpu.VMEM((CB,), jnp.int32),
        pltpu.VMEM((2, KB, 2 * D), jnp.float32),
        pltpu.VMEM((2, KB, 2 * D), jnp.float32),
        pltpu.VMEM((2, KB, 2 * D), jnp.float32),
        pltpu.VMEM((2, KB, 2 * D), jnp.float32),
        pltpu.VMEM((2, KB, 2 * D), jnp.float32),
        pltpu.VMEM((2, KB, 2 * D), jnp.float32),
        pltpu.VMEM((CB,), jnp.float32),
        pltpu.VMEM((CB,), jnp.float32),
        pltpu.VMEM((CB,), jnp.float32),
        pltpu.VMEM((LN,), jnp.int32),
        pltpu.VMEM((LN,), jnp.float32),
        pltpu.SemaphoreType.DMA,
        pltpu.SemaphoreType.DMA,
    ],
)
def _phase_a(emb_h2, emb_t2, bh, bt, br, mk, rel2, ent2, rel_t2,
             pos_hbm, neg_hbm, si_hbm, sf_hbm,
             idxh_v, idxt_v, idxr_v, mask_v, eh_v, et_v, ht_v, tt_v, rr_v, rt_v,
             score_v, pos_v, neg_v, si_v, sf_v, sem0, sem1):
    w = lax.axis_index("s") * NC + lax.axis_index("c")
    base = w * CB
    pltpu.sync_copy(bh.at[pl.ds(base, CB)], idxh_v)
    pltpu.sync_copy(bt.at[pl.ds(base, CB)], idxt_v)
    pltpu.sync_copy(br.at[pl.ds(base, CB)], idxr_v)
    pltpu.sync_copy(mk.at[pl.ds(base, CB)], mask_v)

    iota = lax.iota(jnp.int32, LN)
    sems = (sem0, sem1)
    NB = CB // KB
    GPB = KB // LN

    def copies(blk, p):
        o = blk * KB
        eo = pl.multiple_of(base + o, KB)
        return [
            (ent2.at[idxh_v.at[pl.ds(o, KB)]], ht_v.at[p]),
            (ent2.at[idxt_v.at[pl.ds(o, KB)]], tt_v.at[p]),
            (rel2.at[idxr_v.at[pl.ds(o, KB)]], rr_v.at[p]),
            (rel_t2.at[idxr_v.at[pl.ds(o, KB)]], rt_v.at[p]),
            (emb_h2.at[pl.ds(eo, KB)], eh_v.at[p]),
            (emb_t2.at[pl.ds(eo, KB)], et_v.at[p]),
        ]

    for src, dst in copies(0, 0):
        pltpu.async_copy(src, dst, sem0)

    def body(g, carry):
        blk = g // GPB
        p = blk % 2
        first = (g % GPB) == 0
        for pi in (0, 1):
            @pl.when(first & (p == pi))
            def _(pi=pi, blk=blk):
                for src, dst in copies(blk, pi):
                    pltpu.make_async_copy(src, dst, sems[pi]).wait()

                @pl.when(blk < NB - 1)
                def _(pi=pi, blk=blk):
                    for src, dst in copies(blk + 1, 1 - pi):
                        pltpu.async_copy(src, dst, sems[1 - pi])

        gl = g % GPB
        svec = jnp.zeros((LN,), jnp.float32)
        for k in range(LN):
            i = gl * LN + k
            eh = [eh_v[p, i, pl.ds(LN * c, LN)] for c in range(NV)]
            et = [et_v[p, i, pl.ds(LN * c, LN)] for c in range(NV)]
            ht = [ht_v[p, i, pl.ds(LN * c, LN)] for c in range(NV)]
            tt = [tt_v[p, i, pl.ds(LN * c, LN)] for c in range(NV)]
            rr = [rr_v[p, i, pl.ds(LN * c, LN)] for c in range(NV)]
            rt = [rt_v[p, i, pl.ds(LN * c, LN)] for c in range(NV)]
            dh = _dot_splat(eh, ht)
            dt = _dot_splat(et, tt)
            h = [eh[c] + dh * rt[c] for c in range(NV)]
            t = [et[c] + dt * rt[c] for c in range(NV)]
            ih = _rsqrt(_dot_splat(h, h))
            it = _rsqrt(_dot_splat(t, t))
            ir = _rsqrt(_dot_splat(rr, rr))
            acc = jnp.abs(h[0] * ih + rr[0] * ir - t[0] * it)
            for c in range(1, NV):
                acc = acc + jnp.abs(h[c] * ih + rr[c] * ir - t[c] * it)
            s = jnp.float32(MARGIN) - jnp.sum(acc)
            svec = jnp.where(iota == k, _splat(s), svec)
        plsc.store_scatter(score_v, [_splat(g * LN) + iota], svec)
        return carry

    lax.fori_loop(0, NG, body, jnp.int32(0))

    rank_c = jnp.int32(0)
    accp = jnp.zeros((LN,), jnp.float32)
    accn = jnp.zeros((LN,), jnp.float32)
    for j in range(NG):
        m = mask_v[pl.ds(LN * j, LN)]
        s = score_v[pl.ds(LN * j, LN)]
        cum = plsc.cumsum(m)
        rank_p = _splat(rank_c) + cum - m
        mb = m > 0
        plsc.store_scatter(pos_v, [rank_p], s, mask=mb)
        rank_n = (jnp.int32(LN * j) + iota) - rank_p
        plsc.store_scatter(neg_v, [rank_n], s, mask=jnp.logical_not(mb))
        rank_c = rank_c + cum[LN - 1]
        accp = accp + jnp.where(mb, s, 0.0)
        accn = accn + jnp.where(mb, 0.0, s)

    zi = jnp.zeros((LN,), jnp.int32)
    zf = jnp.zeros((LN,), jnp.float32)
    si_v[...] = jnp.where(iota == 0, _splat(rank_c), zi)
    sf = jnp.where(iota == 0, _splat(jnp.sum(accp)), zf)
    sf_v[...] = jnp.where(iota == 1, _splat(jnp.sum(accn)), sf)
    pltpu.sync_copy(pos_v, pos_hbm.at[pl.ds(base, CB)])
    pltpu.sync_copy(neg_v, neg_hbm.at[pl.ds(base, CB)])
    pltpu.sync_copy(si_v, si_hbm.at[pl.ds(w * LN, LN)])
    pltpu.sync_copy(sf_v, sf_hbm.at[pl.ds(w * LN, LN)])


@functools.partial(
    pl.kernel,
    out_type=jax.ShapeDtypeStruct((NW * LN,), jnp.float32),
    mesh=_mesh,
    compiler_params=_params,
    scratch_types=[
        pltpu.VMEM((B,), jnp.float32),
        pltpu.VMEM((B,), jnp.float32),
        pltpu.VMEM((NW * LN,), jnp.int32),
        pltpu.VMEM((NW * LN,), jnp.float32),
        pltpu.SMEM((NW,), jnp.int32),
        pltpu.SMEM((NW,), jnp.int32),
        pltpu.VMEM((LN,), jnp.float32),
    ],
)
def _phase_b(pos_hbm, neg_hbm, si_hbm, sf_hbm, part_hbm,
             pos_v, neg_v, si_v, sf_v, offp_v, offn_v, stage_v):
    w = lax.axis_index("s") * NC + lax.axis_index("c")
    pltpu.sync_copy(pos_hbm, pos_v)
    pltpu.sync_copy(neg_hbm, neg_v)
    pltpu.sync_copy(si_hbm, si_v)
    pltpu.sync_copy(sf_hbm, sf_v)

    op = jnp.int32(0)
    on = jnp.int32(0)
    accf = sf_v[pl.ds(0, LN)]
    for j in range(NW):
        offp_v[j] = op
        offn_v[j] = on
        cj = si_v[pl.ds(j * LN, LN)][0]
        op = op + cj
        on = on + (jnp.int32(CB) - cj)
        if j > 0:
            accf = accf + sf_v[pl.ds(j * LN, LN)]

    Pv = _splat(op)
    Nv = jnp.int32(B) - Pv
    Lv = jnp.maximum(Pv, Nv)
    mean_p = _splat(accf[0]) / Pv.astype(jnp.float32)
    mean_n = _splat(accf[1]) / Nv.astype(jnp.float32)
    iota = lax.iota(jnp.int32, LN)
    negM = jnp.full((LN,), -MARGIN, jnp.float32)

    def body(v, acc):
        ranks = _splat(w * CB + v * LN) + iota
        selr_p = jnp.zeros((LN,), jnp.int32)
        selo_p = jnp.zeros((LN,), jnp.int32)
        selr_n = jnp.zeros((LN,), jnp.int32)
        selo_n = jnp.zeros((LN,), jnp.int32)
        for j in range(NW):
            oj = _splat(offp_v[j])
            le = oj <= ranks
            selr_p = jnp.where(le, j, selr_p)
            selo_p = jnp.where(le, oj, selo_p)
            oj = _splat(offn_v[j])
            le = oj <= ranks
            selr_n = jnp.where(le, j, selr_n)
            selo_n = jnp.where(le, oj, selo_n)
        colp = jnp.minimum(ranks - selo_p, CB - 1)
        coln = jnp.minimum(ranks - selo_n, CB - 1)
        pv = plsc.load_gather(pos_v, [selr_p * CB + colp])
        nv = plsc.load_gather(neg_v, [selr_n * CB + coln])
        p = jnp.where(ranks < Pv, pv, mean_p)
        n = jnp.where(ranks < Nv, nv, mean_n)
        term = jnp.maximum(p - n, negM)
        return acc + jnp.where(ranks < Lv, term, 0.0)

    acc = lax.fori_loop(0, CB // LN, body, jnp.zeros((LN,), jnp.float32))
    stage_v[...] = acc
    pltpu.sync_copy(stage_v, part_hbm.at[pl.ds(w * LN, LN)])


@functools.partial(
    pl.kernel,
    out_type=jax.ShapeDtypeStruct((LN,), jnp.float32),
    mesh=_mesh,
    compiler_params=_params,
    scratch_types=[
        pltpu.VMEM((NW * LN,), jnp.float32),
        pltpu.VMEM((NW * LN,), jnp.int32),
        pltpu.VMEM((LN,), jnp.float32),
    ],
)
def _phase_c(part_hbm, si_hbm, out_hbm, part_v, si_v, stage_v):
    w = lax.axis_index("s") * NC + lax.axis_index("c")
    pltpu.sync_copy(part_hbm, part_v)
    pltpu.sync_copy(si_hbm, si_v)
    acc = part_v[pl.ds(0, LN)]
    P = si_v[pl.ds(0, LN)][0]
    for j in range(1, NW):
        acc = acc + part_v[pl.ds(j * LN, LN)]
        P = P + si_v[pl.ds(j * LN, LN)][0]
    total = _splat(jnp.sum(acc))
    Pv = _splat(P)
    Lv = jnp.maximum(Pv, jnp.int32(B) - Pv).astype(jnp.float32)
    stage_v[...] = total / Lv + jnp.float32(MARGIN)

    @pl.when(w == 0)
    def _():
        pltpu.sync_copy(stage_v, out_hbm)


def _repack_body(in_ref, out_ref):
    x = in_ref[...]
    z = jnp.zeros((x.shape[1], D), jnp.float32)
    out_ref[...] = jnp.concatenate([x.T.astype(jnp.float32), z], axis=1)


def _repack(table_t, blk):
    m = table_t.shape[1]
    return pl.pallas_call(
        _repack_body,
        grid=(pl.cdiv(m, blk),),
        in_specs=[pl.BlockSpec((D, blk), lambda i: (0, i))],
        out_specs=pl.BlockSpec((blk, 2 * D), lambda i: (i, 0)),
        out_shape=jax.ShapeDtypeStruct((m, 2 * D), jnp.float32),
    )(table_t)


def kernel(emb_h, emb_t, batch_h, batch_t, batch_r, mask,
           rel_embeddings, ent_transfer, rel_transfer):
    bh = batch_h.astype(jnp.int32)
    bt = batch_t.astype(jnp.int32)
    br = batch_r.astype(jnp.int32)
    mk = mask.astype(jnp.int32)
    ent2 = _repack(ent_transfer.T, 16384)
    rel2 = _repack(rel_embeddings.T, 512)
    rel_t2 = _repack(rel_transfer.T, 512)
    emb_h2 = _repack(emb_h.T, 8192)
    emb_t2 = _repack(emb_t.T, 8192)
    pos, neg, si, sf = _phase_a(emb_h2, emb_t2, bh, bt, br, mk,
                                rel2, ent2, rel_t2)
    part = _phase_b(pos, neg, si, sf)
    out16 = _phase_c(part, si)
    return out16[0]


# speedup vs baseline: 6.0919x; 1.0192x over previous
import functools

import jax
import jax.numpy as jnp
from jax import lax
from jax.experimental import pallas as pl
from jax.experimental.pallas import tpu as pltpu
from jax.experimental.pallas import tpu_sc as plsc

B = 16384
D = 64
MARGIN = 4.0
NC = 2
NS = 16
NW = NC * NS
CB = B // NW
KB = 64
LN = 16
NV = 4
NG = CB // LN

_mesh = plsc.VectorSubcoreMesh(
    core_axis_name="c", subcore_axis_name="s", num_cores=NC, num_subcores=NS
)
_params = pltpu.CompilerParams(needs_layout_passes=False)


def _splat(x, dtype=None):
    v = jnp.broadcast_to(x, (LN,))
    return v if dtype is None else v.astype(dtype)


def _rsqrt(x):
    i = plsc.bitcast(x, jnp.int32)
    y = plsc.bitcast(jnp.int32(0x5F3759DF) - (i >> 1), jnp.float32)
    for _ in range(3):
        y = y * (1.5 - 0.5 * x * y * y)
    return y


def _dot_splat(a, b):
    s = a[0] * b[0]
    for c in range(1, NV):
        s = s + a[c] * b[c]
    return _splat(jnp.sum(s))


@functools.partial(
    pl.kernel,
    out_type=[
        jax.ShapeDtypeStruct((B,), jnp.float32),
        jax.ShapeDtypeStruct((B,), jnp.float32),
        jax.ShapeDtypeStruct((NW * LN,), jnp.int32),
        jax.ShapeDtypeStruct((NW * LN,), jnp.float32),
    ],
    mesh=_mesh,
    compiler_params=_params,
    scratch_types=[
        pltpu.VMEM((CB,), jnp.int32),
        pltpu.VMEM((CB,), jnp.int32),
        pltpu.VMEM((CB,), jnp.int32),
        pltpu.VMEM((CB,), jnp.int32),
        pltpu.VMEM((2, KB, 2 * D), jnp.float32),
        pltpu.VMEM((2, KB, 2 * D), jnp.float32),
        pltpu.VMEM((2, KB, 2 * D), jnp.float32),
        pltpu.VMEM((2, KB, 2 * D), jnp.float32),
        pltpu.VMEM((2, KB, 2 * D), jnp.float32),
        pltpu.VMEM((2, KB, 2 * D), jnp.float32),
        pltpu.VMEM((CB,), jnp.float32),
        pltpu.VMEM((CB,), jnp.float32),
        pltpu.VMEM((CB,), jnp.float32),
        pltpu.VMEM((LN,), jnp.int32),
        pltpu.VMEM((LN,), jnp.float32),
        pltpu.SemaphoreType.DMA,
        pltpu.SemaphoreType.DMA,
    ],
)
def _phase_a(emb_h2, emb_t2, bh, bt, br, mk, rel2, ent2, rel_t2,
             pos_hbm, neg_hbm, si_hbm, sf_hbm,
             idxh_v, idxt_v, idxr_v, mask_v, eh_v, et_v, ht_v, tt_v, rr_v, rt_v,
             score_v, pos_v, neg_v, si_v, sf_v, sem0, sem1):
    w = lax.axis_index("s") * NC + lax.axis_index("c")
    base = w * CB
    pltpu.sync_copy(bh.at[pl.ds(base, CB)], idxh_v)
    pltpu.sync_copy(bt.at[pl.ds(base, CB)], idxt_v)
    pltpu.sync_copy(br.at[pl.ds(base, CB)], idxr_v)
    pltpu.sync_copy(mk.at[pl.ds(base, CB)], mask_v)

    iota = lax.iota(jnp.int32, LN)
    sems = (sem0, sem1)
    NB = CB // KB
    GPB = KB // LN

    def copies(blk, p):
        o = blk * KB
        eo = pl.multiple_of(base + o, KB)
        return [
            (ent2.at[idxh_v.at[pl.ds(o, KB)]], ht_v.at[p]),
            (ent2.at[idxt_v.at[pl.ds(o, KB)]], tt_v.at[p]),
            (rel2.at[idxr_v.at[pl.ds(o, KB)]], rr_v.at[p]),
            (rel_t2.at[idxr_v.at[pl.ds(o, KB)]], rt_v.at[p]),
            (emb_h2.at[pl.ds(eo, KB)], eh_v.at[p]),
            (emb_t2.at[pl.ds(eo, KB)], et_v.at[p]),
        ]

    for src, dst in copies(0, 0):
        pltpu.async_copy(src, dst, sem0)

    def body(g, carry):
        blk = g // GPB
        p = blk % 2
        first = (g % GPB) == 0
        for pi in (0, 1):
            @pl.when(first & (p == pi))
            def _(pi=pi, blk=blk):
                for src, dst in copies(blk, pi):
                    pltpu.make_async_copy(src, dst, sems[pi]).wait()

                @pl.when(blk < NB - 1)
                def _(pi=pi, blk=blk):
                    for src, dst in copies(blk + 1, 1 - pi):
                        pltpu.async_copy(src, dst, sems[1 - pi])

        gl = g % GPB
        svec = jnp.zeros((LN,), jnp.float32)
        for k in range(LN):
            i = gl * LN + k
            eh = [eh_v[p, i, pl.ds(LN * c, LN)] for c in range(NV)]
            et = [et_v[p, i, pl.ds(LN * c, LN)] for c in range(NV)]
            ht = [ht_v[p, i, pl.ds(LN * c, LN)] for c in range(NV)]
            tt = [tt_v[p, i, pl.ds(LN * c, LN)] for c in range(NV)]
            rr = [rr_v[p, i, pl.ds(LN * c, LN)] for c in range(NV)]
            rt = [rt_v[p, i, pl.ds(LN * c, LN)] for c in range(NV)]
            dh = _dot_splat(eh, ht)
            dt = _dot_splat(et, tt)
            h = [eh[c] + dh * rt[c] for c in range(NV)]
            t = [et[c] + dt * rt[c] for c in range(NV)]
            ih = _rsqrt(_dot_splat(h, h))
            it = _rsqrt(_dot_splat(t, t))
            ir = _rsqrt(_dot_splat(rr, rr))
            acc = jnp.abs(h[0] * ih + rr[0] * ir - t[0] * it)
            for c in range(1, NV):
                acc = acc + jnp.abs(h[c] * ih + rr[c] * ir - t[c] * it)
            s = jnp.float32(MARGIN) - jnp.sum(acc)
            svec = jnp.where(iota == k, _splat(s), svec)
        plsc.store_scatter(score_v, [_splat(g * LN) + iota], svec)
        return carry

    lax.fori_loop(0, NG, body, jnp.int32(0))

    rank_c = jnp.int32(0)
    accp = jnp.zeros((LN,), jnp.float32)
    accn = jnp.zeros((LN,), jnp.float32)
    for j in range(NG):
        m = mask_v[pl.ds(LN * j, LN)]
        s = score_v[pl.ds(LN * j, LN)]
        cum = plsc.cumsum(m)
        rank_p = _splat(rank_c) + cum - m
        mb = m > 0
        plsc.store_scatter(pos_v, [rank_p], s, mask=mb)
        rank_n = (jnp.int32(LN * j) + iota) - rank_p
        plsc.store_scatter(neg_v, [rank_n], s, mask=jnp.logical_not(mb))
        rank_c = rank_c + cum[LN - 1]
        accp = accp + jnp.where(mb, s, 0.0)
        accn = accn + jnp.where(mb, 0.0, s)

    zi = jnp.zeros((LN,), jnp.int32)
    zf = jnp.zeros((LN,), jnp.float32)
    si_v[...] = jnp.where(iota == 0, _splat(rank_c), zi)
    sf = jnp.where(iota == 0, _splat(jnp.sum(accp)), zf)
    sf_v[...] = jnp.where(iota == 1, _splat(jnp.sum(accn)), sf)
    pltpu.sync_copy(pos_v, pos_hbm.at[pl.ds(base, CB)])
    pltpu.sync_copy(neg_v, neg_hbm.at[pl.ds(base, CB)])
    pltpu.sync_copy(si_v, si_hbm.at[pl.ds(w * LN, LN)])
    pltpu.sync_copy(sf_v, sf_hbm.at[pl.ds(w * LN, LN)])


@functools.partial(
    pl.kernel,
    out_type=jax.ShapeDtypeStruct((NW * LN,), jnp.float32),
    mesh=_mesh,
    compiler_params=_params,
    scratch_types=[
        pltpu.VMEM((B,), jnp.float32),
        pltpu.VMEM((B,), jnp.float32),
        pltpu.VMEM((NW * LN,), jnp.int32),
        pltpu.VMEM((NW * LN,), jnp.float32),
        pltpu.SMEM((NW,), jnp.int32),
        pltpu.SMEM((NW,), jnp.int32),
        pltpu.VMEM((LN,), jnp.float32),
    ],
)
def _phase_b(pos_hbm, neg_hbm, si_hbm, sf_hbm, part_hbm,
             pos_v, neg_v, si_v, sf_v, offp_v, offn_v, stage_v):
    w = lax.axis_index("s") * NC + lax.axis_index("c")
    pltpu.sync_copy(pos_hbm, pos_v)
    pltpu.sync_copy(neg_hbm, neg_v)
    pltpu.sync_copy(si_hbm, si_v)
    pltpu.sync_copy(sf_hbm, sf_v)

    op = jnp.int32(0)
    on = jnp.int32(0)
    accf = sf_v[pl.ds(0, LN)]
    for j in range(NW):
        offp_v[j] = op
        offn_v[j] = on
        cj = si_v[pl.ds(j * LN, LN)][0]
        op = op + cj
        on = on + (jnp.int32(CB) - cj)
        if j > 0:
            accf = accf + sf_v[pl.ds(j * LN, LN)]

    Pv = _splat(op)
    Nv = jnp.int32(B) - Pv
    Lv = jnp.maximum(Pv, Nv)
    mean_p = _splat(accf[0]) / Pv.astype(jnp.float32)
    mean_n = _splat(accf[1]) / Nv.astype(jnp.float32)
    iota = lax.iota(jnp.int32, LN)
    negM = jnp.full((LN,), -MARGIN, jnp.float32)

    def body(v, acc):
        ranks = _splat(w * CB + v * LN) + iota
        selr_p = jnp.zeros((LN,), jnp.int32)
        selo_p = jnp.zeros((LN,), jnp.int32)
        selr_n = jnp.zeros((LN,), jnp.int32)
        selo_n = jnp.zeros((LN,), jnp.int32)
        for j in range(NW):
            oj = _splat(offp_v[j])
            le = oj <= ranks
            selr_p = jnp.where(le, j, selr_p)
            selo_p = jnp.where(le, oj, selo_p)
            oj = _splat(offn_v[j])
            le = oj <= ranks
            selr_n = jnp.where(le, j, selr_n)
            selo_n = jnp.where(le, oj, selo_n)
        colp = jnp.minimum(ranks - selo_p, CB - 1)
        coln = jnp.minimum(ranks - selo_n, CB - 1)
        pv = plsc.load_gather(pos_v, [selr_p * CB + colp])
        nv = plsc.load_gather(neg_v, [selr_n * CB + coln])
        p = jnp.where(ranks < Pv, pv, mean_p)
        n = jnp.where(ranks < Nv, nv, mean_n)
        term = jnp.maximum(p - n, negM)
        return acc + jnp.where(ranks < Lv, term, 0.0)

    acc = lax.fori_loop(0, CB // LN, body, jnp.zeros((LN,), jnp.float32))
    stage_v[...] = acc
    pltpu.sync_copy(stage_v, part_hbm.at[pl.ds(w * LN, LN)])


@functools.partial(
    pl.kernel,
    out_type=jax.ShapeDtypeStruct((LN,), jnp.float32),
    mesh=_mesh,
    compiler_params=_params,
    scratch_types=[
        pltpu.VMEM((NW * LN,), jnp.float32),
        pltpu.VMEM((NW * LN,), jnp.int32),
        pltpu.VMEM((LN,), jnp.float32),
    ],
)
def _phase_c(part_hbm, si_hbm, out_hbm, part_v, si_v, stage_v):
    w = lax.axis_index("s") * NC + lax.axis_index("c")
    pltpu.sync_copy(part_hbm, part_v)
    pltpu.sync_copy(si_hbm, si_v)
    acc = part_v[pl.ds(0, LN)]
    P = si_v[pl.ds(0, LN)][0]
    for j in range(1, NW):
        acc = acc + part_v[pl.ds(j * LN, LN)]
        P = P + si_v[pl.ds(j * LN, LN)][0]
    total = _splat(jnp.sum(acc))
    Pv = _splat(P)
    Lv = jnp.maximum(Pv, jnp.int32(B) - Pv).astype(jnp.float32)
    stage_v[...] = total / Lv + jnp.float32(MARGIN)

    @pl.when(w == 0)
    def _():
        pltpu.sync_copy(stage_v, out_hbm)


def _repack_body(in_ref, out_ref):
    x = in_ref[...]
    z = jnp.zeros((x.shape[1], D), jnp.float32)
    out_ref[...] = jnp.concatenate([x.T.astype(jnp.float32), z], axis=1)


def _repack(table_t, blk):
    m = table_t.shape[1]
    return pl.pallas_call(
        _repack_body,
        grid=(pl.cdiv(m, blk),),
        in_specs=[pl.BlockSpec((D, blk), lambda i: (0, i))],
        out_specs=pl.BlockSpec((blk, 2 * D), lambda i: (i, 0)),
        out_shape=jax.ShapeDtypeStruct((m, 2 * D), jnp.float32),
    )(table_t)


def kernel(emb_h, emb_t, batch_h, batch_t, batch_r, mask,
           rel_embeddings, ent_transfer, rel_transfer):
    bh = batch_h.astype(jnp.int32)
    bt = batch_t.astype(jnp.int32)
    br = batch_r.astype(jnp.int32)
    mk = mask.astype(jnp.int32)
    ent2 = _repack(ent_transfer.T, 32768)
    rel2 = _repack(rel_embeddings.T, 512)
    rel_t2 = _repack(rel_transfer.T, 512)
    emb_h2 = _repack(emb_h.T, 8192)
    emb_t2 = _repack(emb_t.T, 8192)
    pos, neg, si, sf = _phase_a(emb_h2, emb_t2, bh, bt, br, mk,
                                rel2, ent2, rel_t2)
    part = _phase_b(pos, neg, si, sf)
    out16 = _phase_c(part, si)
    return out16[0]
